# baseline (speedup 1.0000x reference)
import functools

import jax
import jax.numpy as jnp
from jax.experimental import pallas as pl
from jax.experimental.pallas import tpu as pltpu

_BLOCK_ROWS = 256


def _calib_kernel(logits_ref, hist_ref, out_ref, valid_ref, pred_ref):
    x = logits_ref[...]
    br, c = x.shape
    nb = hist_ref.shape[1]

    m = jnp.max(x, axis=1, keepdims=True)
    e = jnp.exp(x - m)
    s = jnp.sum(e, axis=1, keepdims=True)
    conf = 1.0 / s

    cols = jax.lax.broadcasted_iota(jnp.int32, (br, c), 1)
    pred = jnp.min(jnp.where(e == 1.0, cols, c), axis=1, keepdims=True)

    bin_idx = jnp.clip(jnp.ceil(conf * nb).astype(jnp.int32) - 1, 0, nb - 1)
    bins = jax.lax.broadcasted_iota(jnp.int32, (br, nb), 1)
    hist = hist_ref[...]
    h = jnp.sum(jnp.where(bin_idx == bins, hist, 0.0), axis=1, keepdims=True)

    valid = (h != -1.0).astype(jnp.float32)
    est = jnp.where(h == -1.0, conf, h)

    onehot = cols == pred
    masked = jnp.where(onehot, 0.0, e)
    denom = jnp.sum(masked, axis=1, keepdims=True)
    scale = (1.0 - est) / denom
    rescaled = jnp.where(onehot, est, masked * scale)
    total = jnp.sum(rescaled, axis=1, keepdims=True)
    out_ref[...] = rescaled * (1.0 / total)
    valid_ref[...] = valid
    pred_ref[...] = pred


@functools.partial(jax.jit, static_argnames=())
def kernel(logits, histogram):
    n, c = logits.shape
    nb = histogram.shape[0]
    br = _BLOCK_ROWS
    grid = (n // br,)
    hist2d = histogram.reshape(1, nb)
    sm_calib, valid, pred = pl.pallas_call(
        _calib_kernel,
        grid=grid,
        in_specs=[
            pl.BlockSpec((br, c), lambda i: (i, 0)),
            pl.BlockSpec((1, nb), lambda i: (0, 0)),
        ],
        out_specs=[
            pl.BlockSpec((br, c), lambda i: (i, 0)),
            pl.BlockSpec((br, 1), lambda i: (i, 0)),
            pl.BlockSpec((br, 1), lambda i: (i, 0)),
        ],
        out_shape=[
            jax.ShapeDtypeStruct((n, c), jnp.float32),
            jax.ShapeDtypeStruct((n, 1), jnp.float32),
            jax.ShapeDtypeStruct((n, 1), jnp.int32),
        ],
        compiler_params=pltpu.CompilerParams(
            dimension_semantics=("arbitrary",),
        ),
        name="hist_binning_calib",
    )(logits, hist2d)
    return sm_calib, valid.reshape(n), pred.reshape(n)


# speedup vs baseline: 3.2896x; 3.2896x over previous
import functools

import jax
import jax.numpy as jnp
from jax.experimental import pallas as pl
from jax.experimental.pallas import tpu as pltpu

_BLOCK_COLS = 512


def _calib_kernel(x_ref, hist_ref, out_ref, valid_ref, pred_ref):
    x = x_ref[...]
    c, bn = x.shape
    nb = hist_ref.shape[0]

    m = jnp.max(x, axis=0, keepdims=True)
    e = jnp.exp(x - m)
    s = jnp.sum(e, axis=0, keepdims=True)
    conf = 1.0 / s

    rows = jax.lax.broadcasted_iota(jnp.int32, (c, bn), 0)
    pred = jnp.min(jnp.where(e == 1.0, rows, c), axis=0, keepdims=True)

    bin_idx = jnp.clip(jnp.ceil(conf * nb).astype(jnp.int32) - 1, 0, nb - 1)
    h = jnp.full_like(conf, hist_ref[0])
    for b in range(1, nb):
        h = jnp.where(bin_idx == b, hist_ref[b], h)

    valid = (h != -1.0).astype(jnp.float32)
    est = jnp.where(h == -1.0, conf, h)

    onehot = rows == pred
    masked = jnp.where(onehot, 0.0, e)
    denom = jnp.sum(masked, axis=0, keepdims=True)
    scale = (1.0 - est) / denom
    rescaled = jnp.where(onehot, est, masked * scale)
    total = jnp.sum(rescaled, axis=0, keepdims=True)
    out_ref[...] = rescaled * (1.0 / total)
    valid_ref[...] = valid
    pred_ref[...] = pred


@jax.jit
def kernel(logits, histogram):
    n, c = logits.shape
    nb = histogram.shape[0]
    bn = _BLOCK_COLS
    xt = logits.T
    out_t, valid, pred = pl.pallas_call(
        _calib_kernel,
        grid=(n // bn,),
        in_specs=[
            pl.BlockSpec((c, bn), lambda i: (0, i)),
            pl.BlockSpec(memory_space=pltpu.SMEM),
        ],
        out_specs=[
            pl.BlockSpec((c, bn), lambda i: (0, i)),
            pl.BlockSpec((1, bn), lambda i: (0, i)),
            pl.BlockSpec((1, bn), lambda i: (0, i)),
        ],
        out_shape=[
            jax.ShapeDtypeStruct((c, n), jnp.float32),
            jax.ShapeDtypeStruct((1, n), jnp.float32),
            jax.ShapeDtypeStruct((1, n), jnp.int32),
        ],
        compiler_params=pltpu.CompilerParams(
            dimension_semantics=("arbitrary",),
        ),
        name="hist_binning_calib",
    )(xt, histogram)
    return out_t.T, valid.reshape(n), pred.reshape(n)


# speedup vs baseline: 3.7226x; 1.1316x over previous
import functools

import jax
import jax.numpy as jnp
from jax.experimental import pallas as pl
from jax.experimental.pallas import tpu as pltpu

_BLOCK_COLS = 512


def _calib_kernel(x_ref, hist_ref, out_ref, valid_ref, pred_ref):
    x = x_ref[...]
    c, bn = x.shape
    nb = hist_ref.shape[0]

    m = jnp.max(x, axis=0, keepdims=True)
    e = jnp.exp(x - m)
    s = jnp.sum(e, axis=0, keepdims=True)
    conf = 1.0 / s

    rows = jax.lax.broadcasted_iota(jnp.int32, (c, bn), 0)
    pred = jnp.min(jnp.where(e == 1.0, rows, c), axis=0, keepdims=True)

    bin_idx = jnp.clip(jnp.ceil(conf * nb).astype(jnp.int32) - 1, 0, nb - 1)
    h = jnp.full_like(conf, hist_ref[0])
    for b in range(1, nb):
        h = jnp.where(bin_idx == b, hist_ref[b], h)

    valid = (h != -1.0).astype(jnp.float32)
    est = jnp.where(h == -1.0, conf, h)

    scale = (1.0 - est) / (s - 1.0)
    out_ref[...] = jnp.where(rows == pred, est, e * scale)
    valid_ref[...] = valid
    pred_ref[...] = pred


@jax.jit
def kernel(logits, histogram):
    n, c = logits.shape
    nb = histogram.shape[0]
    bn = _BLOCK_COLS
    xt = logits.T
    out_t, valid, pred = pl.pallas_call(
        _calib_kernel,
        grid=(n // bn,),
        in_specs=[
            pl.BlockSpec((c, bn), lambda i: (0, i)),
            pl.BlockSpec(memory_space=pltpu.SMEM),
        ],
        out_specs=[
            pl.BlockSpec((c, bn), lambda i: (0, i)),
            pl.BlockSpec((1, bn), lambda i: (0, i)),
            pl.BlockSpec((1, bn), lambda i: (0, i)),
        ],
        out_shape=[
            jax.ShapeDtypeStruct((c, n), jnp.float32),
            jax.ShapeDtypeStruct((1, n), jnp.float32),
            jax.ShapeDtypeStruct((1, n), jnp.int32),
        ],
        compiler_params=pltpu.CompilerParams(
            dimension_semantics=("arbitrary",),
        ),
        name="hist_binning_calib",
    )(xt, histogram)
    return out_t.T, valid.reshape(n), pred.reshape(n)


# speedup vs baseline: 4.4122x; 1.1853x over previous
import functools

import jax
import jax.numpy as jnp
from jax.experimental import pallas as pl
from jax.experimental.pallas import tpu as pltpu

_BLOCK_COLS = 1024


def _calib_kernel(x_ref, hist_ref, out_ref, valid_ref, pred_ref):
    x = x_ref[...]
    c, bn = x.shape
    nb = hist_ref.shape[0]

    m = jnp.max(x, axis=0, keepdims=True)
    e = jnp.exp(x - m)
    s = jnp.sum(e, axis=0, keepdims=True)
    conf = 1.0 / s

    rows = jax.lax.broadcasted_iota(jnp.int32, (c, bn), 0)
    pred = jnp.min(jnp.where(e == 1.0, rows, c), axis=0, keepdims=True)

    bin_idx = jnp.clip(jnp.ceil(conf * nb).astype(jnp.int32) - 1, 0, nb - 1)
    h = jnp.full_like(conf, hist_ref[0])
    for b in range(1, nb):
        h = jnp.where(bin_idx == b, hist_ref[b], h)

    valid = (h != -1.0).astype(jnp.float32)
    est = jnp.where(h == -1.0, conf, h)

    scale = (1.0 - est) / (s - 1.0)
    out_ref[...] = jnp.where(rows == pred, est, e * scale)
    valid_ref[...] = valid
    pred_ref[...] = pred


@jax.jit
def kernel(logits, histogram):
    n, c = logits.shape
    nb = histogram.shape[0]
    bn = _BLOCK_COLS
    xt = logits.T
    out_t, valid, pred = pl.pallas_call(
        _calib_kernel,
        grid=(n // bn,),
        in_specs=[
            pl.BlockSpec((c, bn), lambda i: (0, i)),
            pl.BlockSpec(memory_space=pltpu.SMEM),
        ],
        out_specs=[
            pl.BlockSpec((c, bn), lambda i: (0, i)),
            pl.BlockSpec((1, bn), lambda i: (0, i)),
            pl.BlockSpec((1, bn), lambda i: (0, i)),
        ],
        out_shape=[
            jax.ShapeDtypeStruct((c, n), jnp.float32),
            jax.ShapeDtypeStruct((1, n), jnp.float32),
            jax.ShapeDtypeStruct((1, n), jnp.int32),
        ],
        compiler_params=pltpu.CompilerParams(
            dimension_semantics=("arbitrary",),
        ),
        name="hist_binning_calib",
    )(xt, histogram)
    return out_t.T, valid.reshape(n), pred.reshape(n)


# speedup vs baseline: 4.6524x; 1.0544x over previous
import functools

import jax
import jax.numpy as jnp
from jax.experimental import pallas as pl
from jax.experimental.pallas import tpu as pltpu

_BLOCK_COLS = 2048


def _calib_kernel(x_ref, hist_ref, out_ref, valid_ref, pred_ref):
    x = x_ref[...]
    c, bn = x.shape
    nb = hist_ref.shape[0]

    m = jnp.max(x, axis=0, keepdims=True)
    e = jnp.exp(x - m)
    s = jnp.sum(e, axis=0, keepdims=True)
    conf = 1.0 / s

    rows = jax.lax.broadcasted_iota(jnp.int32, (c, bn), 0)
    pred = jnp.min(jnp.where(e == 1.0, rows, c), axis=0, keepdims=True)

    bin_idx = jnp.clip(jnp.ceil(conf * nb).astype(jnp.int32) - 1, 0, nb - 1)
    h = jnp.full_like(conf, hist_ref[0])
    for b in range(1, nb):
        h = jnp.where(bin_idx == b, hist_ref[b], h)

    valid = (h != -1.0).astype(jnp.float32)
    est = jnp.where(h == -1.0, conf, h)

    scale = (1.0 - est) / (s - 1.0)
    out_ref[...] = jnp.where(rows == pred, est, e * scale)
    valid_ref[...] = valid
    pred_ref[...] = pred


@jax.jit
def kernel(logits, histogram):
    n, c = logits.shape
    nb = histogram.shape[0]
    bn = _BLOCK_COLS
    xt = logits.T
    out_t, valid, pred = pl.pallas_call(
        _calib_kernel,
        grid=(n // bn,),
        in_specs=[
            pl.BlockSpec((c, bn), lambda i: (0, i)),
            pl.BlockSpec(memory_space=pltpu.SMEM),
        ],
        out_specs=[
            pl.BlockSpec((c, bn), lambda i: (0, i)),
            pl.BlockSpec((1, bn), lambda i: (0, i)),
            pl.BlockSpec((1, bn), lambda i: (0, i)),
        ],
        out_shape=[
            jax.ShapeDtypeStruct((c, n), jnp.float32),
            jax.ShapeDtypeStruct((1, n), jnp.float32),
            jax.ShapeDtypeStruct((1, n), jnp.int32),
        ],
        compiler_params=pltpu.CompilerParams(
            dimension_semantics=("arbitrary",),
            vmem_limit_bytes=56 * 1024 * 1024,
        ),
        name="hist_binning_calib",
    )(xt, histogram)
    return out_t.T, valid.reshape(n), pred.reshape(n)
